# speedup vs baseline: 3.5302x; 3.5302x over previous
import functools

import jax
import jax.numpy as jnp
from jax import lax
from jax.experimental import pallas as pl
from jax.experimental.pallas import tpu as pltpu
from jax.experimental.pallas import tpu_sc as plsc

_VOCAB = 100000
_SEQ = 2048
_EMBED = 128
_BATCH = 32

_NC = 2
_NS = 16
_NW = _NC * _NS
_CHUNK = 512


def _sc_gather(ids_flat, table):
    n = ids_flat.shape[0]
    per_w = n // _NW
    mesh = plsc.VectorSubcoreMesh(core_axis_name="c", subcore_axis_name="s")

    @functools.partial(
        pl.kernel,
        mesh=mesh,
        out_type=jax.ShapeDtypeStruct((n, _EMBED), jnp.float32),
        scratch_types=[
            pltpu.VMEM((_CHUNK,), jnp.int32),
            pltpu.VMEM((_CHUNK, _EMBED), jnp.float32),
            pltpu.SemaphoreType.DMA,
        ],
    )
    def k(ids_hbm, table_hbm, out_hbm, idx_v, rows_v, sem):
        wid = lax.axis_index("s") * _NC + lax.axis_index("c")
        base = wid * per_w
        for c in range(per_w // _CHUNK):
            off = base + c * _CHUNK
            pltpu.sync_copy(ids_hbm.at[pl.ds(off, _CHUNK)], idx_v)
            pltpu.async_copy(table_hbm.at[idx_v], rows_v, sem).wait()
            pltpu.sync_copy(rows_v, out_hbm.at[pl.ds(off, _CHUNK)])

    return k(ids_flat, table)


def _tc_body(tok_ref, seg_ref, pos_ref, st_ref, g_ref, b_ref, out_ref):
    x = tok_ref[0] + pos_ref[...]
    segc = jnp.reshape(seg_ref[0], (_SEQ, 1))
    x = x + st_ref[0:1, :] + segc * (st_ref[1:2, :] - st_ref[0:1, :])
    mu = jnp.mean(x, axis=-1, keepdims=True)
    xm = x - mu
    var = jnp.mean(xm * xm, axis=-1, keepdims=True)
    out_ref[0] = xm * lax.rsqrt(var + 1e-5) * g_ref[...] + b_ref[...]


def _tc_finish(tok_rows, seg_f, position_table, segment_table, g2, b2):
    return pl.pallas_call(
        _tc_body,
        grid=(_BATCH,),
        in_specs=[
            pl.BlockSpec((1, _SEQ, _EMBED), lambda i: (i, 0, 0)),
            pl.BlockSpec((1, 1, _SEQ), lambda i: (i, 0, 0)),
            pl.BlockSpec((_SEQ, _EMBED), lambda i: (0, 0)),
            pl.BlockSpec((2, _EMBED), lambda i: (0, 0)),
            pl.BlockSpec((1, _EMBED), lambda i: (0, 0)),
            pl.BlockSpec((1, _EMBED), lambda i: (0, 0)),
        ],
        out_specs=pl.BlockSpec((1, _SEQ, _EMBED), lambda i: (i, 0, 0)),
        out_shape=jax.ShapeDtypeStruct((_BATCH, _SEQ, _EMBED), jnp.float32),
    )(tok_rows, seg_f, position_table, segment_table, g2, b2)


def kernel(input_ids, segment_ids, token_table, position_table, segment_table,
           ln_gamma, ln_beta):
    ids_flat = jnp.reshape(input_ids.astype(jnp.int32), (-1,))
    tok_rows = _sc_gather(ids_flat, token_table)
    tok_rows = jnp.reshape(tok_rows, (_BATCH, _SEQ, _EMBED))
    seg_f = jnp.reshape(segment_ids.astype(jnp.float32), (_BATCH, 1, _SEQ))
    g2 = jnp.reshape(ln_gamma, (1, _EMBED))
    b2 = jnp.reshape(ln_beta, (1, _EMBED))
    return _tc_finish(tok_rows, seg_f, position_table, segment_table, g2, b2)
